# Initial kernel scaffold; baseline (speedup 1.0000x reference)
import functools

import jax
import jax.numpy as jnp
from jax import lax
from jax.experimental import pallas as pl
from jax.experimental.pallas import tpu as pltpu
from jax.experimental.pallas import tpu_sc as plsc

B, L, V, D = 4096, 200, 1000000, 64
N = B * L
NW = 32
PER_W = N // NW
IDXW = 128
CHUNK_IR = 8
CHUNK = CHUNK_IR * IDXW
NCHUNK = PER_W // CHUNK
IR_PER_W = PER_W // IDXW
SCALE = float(D) ** 0.5
LANES = 16


def _sc_body(table, seq2d, seq1d, pos2d, pos1d, neg2d, neg1d,
             seq_out, pos_out, neg_out, ist_out,
             idx_v, idxf_v, rows_v, ist_v, sem):
    wid = lax.axis_index("s") * 2 + lax.axis_index("c")

    def run_array(idx2d, idx1d, out_hbm, scaled, want_ist):
        def chunk_body(g, carry):
            ir = wid * IR_PER_W + g * CHUNK_IR
            base = wid * PER_W + g * CHUNK
            pltpu.sync_copy(idx2d.at[pl.ds(ir, CHUNK_IR)], idx_v)
            pltpu.sync_copy(idx1d.at[pl.ds(base, CHUNK)], idxf_v)
            copies = [
                pltpu.async_copy(
                    table.at[idx_v.at[j]],
                    rows_v.at[pl.ds(j * IDXW, IDXW)],
                    sem,
                )
                for j in range(CHUNK_IR)
            ]
            for c in copies:
                c.wait()

            if scaled:
                def scale_row(r, c2):
                    for cc in range(D // LANES):
                        sl = (r, pl.ds(cc * LANES, LANES))
                        rows_v[sl] = rows_v[sl] * SCALE
                    return c2
                lax.fori_loop(0, CHUNK, scale_row, 0)

            def group_body(k, c2):
                gbase = k * LANES
                v = idxf_v[pl.ds(gbase, LANES)]
                if want_ist:
                    ist_v[pl.ds(gbase, LANES)] = jnp.where(
                        v != 0, jnp.float32(1.0), jnp.float32(0.0))

                @pl.when(jnp.any(v == 0))
                def _zero_fix():
                    for lane in range(LANES):
                        s = idxf_v[gbase + lane]

                        @pl.when(s == 0)
                        def _zero_row():
                            for cc in range(D // LANES):
                                rows_v[gbase + lane,
                                       pl.ds(cc * LANES, LANES)] = (
                                    jnp.zeros((LANES,), jnp.float32))
                return c2
            lax.fori_loop(0, CHUNK // LANES, group_body, 0)

            pltpu.sync_copy(rows_v, out_hbm.at[pl.ds(base, CHUNK)])
            if want_ist:
                pltpu.sync_copy(ist_v, ist_out.at[pl.ds(base, CHUNK)])
            return carry

        lax.fori_loop(0, NCHUNK, chunk_body, 0)

    run_array(seq2d, seq1d, seq_out, True, False)
    run_array(pos2d, pos1d, pos_out, False, True)
    run_array(neg2d, neg1d, neg_out, False, False)


@jax.jit
def _sc_call(table, seq2d, seq1d, pos2d, pos1d, neg2d, neg1d):
    mesh = plsc.VectorSubcoreMesh(core_axis_name="c", subcore_axis_name="s")
    f = pl.kernel(
        _sc_body,
        out_type=(
            jax.ShapeDtypeStruct((N, D), jnp.float32),
            jax.ShapeDtypeStruct((N, D), jnp.float32),
            jax.ShapeDtypeStruct((N, D), jnp.float32),
            jax.ShapeDtypeStruct((N,), jnp.float32),
        ),
        mesh=mesh,
        scratch_types=[
            pltpu.VMEM((CHUNK_IR, IDXW), jnp.int32),
            pltpu.VMEM((CHUNK,), jnp.int32),
            pltpu.VMEM((CHUNK, D), jnp.float32),
            pltpu.VMEM((CHUNK,), jnp.float32),
            pltpu.SemaphoreType.DMA,
        ],
    )
    return f(table, seq2d, seq1d, pos2d, pos1d, neg2d, neg1d)


def kernel(seq_ids, pos_ids, neg_ids, item_embedding_table):
    seq2d = seq_ids.reshape(N // IDXW, IDXW)
    pos2d = pos_ids.reshape(N // IDXW, IDXW)
    neg2d = neg_ids.reshape(N // IDXW, IDXW)
    seq1d = seq_ids.reshape(N)
    pos1d = pos_ids.reshape(N)
    neg1d = neg_ids.reshape(N)
    seq_emb, pos_emb, neg_emb, istarget = _sc_call(
        item_embedding_table, seq2d, seq1d, pos2d, pos1d, neg2d, neg1d)
    return seq_emb, pos_emb, neg_emb, istarget


# speedup vs baseline: 1.1088x; 1.1088x over previous
import jax
import jax.numpy as jnp
from jax import lax
from jax.experimental import pallas as pl
from jax.experimental.pallas import tpu as pltpu
from jax.experimental.pallas import tpu_sc as plsc

B, L, V, D = 4096, 200, 1000000, 64
N = B * L
NW = 32
PER_W = N // NW
IDXW = 128
CHUNK_IR = 8
CHUNK = CHUNK_IR * IDXW
NCHUNK = PER_W // CHUNK
IR_PER_W = PER_W // IDXW
SCALE = float(D) ** 0.5
LANES = 16
GPR = IDXW // LANES


def _sc_body(table, seq2d, pos2d, neg2d,
             seq_out, pos_out, neg_out, ist_out,
             idx_v, rows_v, ist_v, zacc_v, sem):
    wid = lax.axis_index("s") * 2 + lax.axis_index("c")

    def load_group(k):
        j = k // GPR
        off = (k % GPR) * LANES
        return idx_v[j, pl.ds(off, LANES)]

    def run_array(idx2d, out_hbm, scaled, want_ist):
        def chunk_body(g, carry):
            ir = wid * IR_PER_W + g * CHUNK_IR
            base = wid * PER_W + g * CHUNK
            pltpu.sync_copy(idx2d.at[pl.ds(ir, CHUNK_IR)], idx_v)
            copies = [
                pltpu.async_copy(
                    table.at[idx_v.at[j]],
                    rows_v.at[pl.ds(j * IDXW, IDXW)],
                    sem,
                )
                for j in range(CHUNK_IR)
            ]
            for c in copies:
                c.wait()

            if scaled:
                def scale_group(k, c2):
                    gbase = k * LANES
                    v = load_group(k)
                    m = jnp.where(v != 0, jnp.float32(SCALE),
                                  jnp.float32(0.0))
                    for lane in range(LANES):
                        s = m[lane]
                        for cc in range(D // LANES):
                            sl = (gbase + lane, pl.ds(cc * LANES, LANES))
                            rows_v[sl] = rows_v[sl] * s
                    return c2
                lax.fori_loop(0, CHUNK // LANES, scale_group, 0)
            else:
                zacc_v[...] = jnp.zeros((LANES,), jnp.int32)

                def group_body(k, c2):
                    v = load_group(k)
                    z = jnp.where(v == 0, jnp.int32(1), jnp.int32(0))
                    zacc_v[...] = zacc_v[...] | z
                    if want_ist:
                        ist_v[pl.ds(k * LANES, LANES)] = jnp.where(
                            v != 0, jnp.float32(1.0), jnp.float32(0.0))
                    return c2
                lax.fori_loop(0, CHUNK // LANES, group_body, 0)

                zacc = zacc_v[...]
                anyz = zacc[0]
                for lane in range(1, LANES):
                    anyz = anyz | zacc[lane]

                @pl.when(anyz != 0)
                def _zero_fix():
                    def fix_group(k, c2):
                        gbase = k * LANES
                        v = load_group(k)
                        for lane in range(LANES):
                            @pl.when(v[lane] == 0)
                            def _zero_row():
                                for cc in range(D // LANES):
                                    rows_v[gbase + lane,
                                           pl.ds(cc * LANES, LANES)] = (
                                        jnp.zeros((LANES,), jnp.float32))
                        return c2
                    lax.fori_loop(0, CHUNK // LANES, fix_group, 0)

            pltpu.sync_copy(rows_v, out_hbm.at[pl.ds(base, CHUNK)])
            if want_ist:
                pltpu.sync_copy(ist_v, ist_out.at[pl.ds(base, CHUNK)])
            return carry

        lax.fori_loop(0, NCHUNK, chunk_body, 0)

    run_array(seq2d, seq_out, True, False)
    run_array(pos2d, pos_out, False, True)
    run_array(neg2d, neg_out, False, False)


@jax.jit
def _sc_call(table, seq2d, pos2d, neg2d):
    mesh = plsc.VectorSubcoreMesh(core_axis_name="c", subcore_axis_name="s")
    f = pl.kernel(
        _sc_body,
        out_type=(
            jax.ShapeDtypeStruct((N, D), jnp.float32),
            jax.ShapeDtypeStruct((N, D), jnp.float32),
            jax.ShapeDtypeStruct((N, D), jnp.float32),
            jax.ShapeDtypeStruct((N,), jnp.float32),
        ),
        mesh=mesh,
        scratch_types=[
            pltpu.VMEM((CHUNK_IR, IDXW), jnp.int32),
            pltpu.VMEM((CHUNK, D), jnp.float32),
            pltpu.VMEM((CHUNK,), jnp.float32),
            pltpu.VMEM((LANES,), jnp.int32),
            pltpu.SemaphoreType.DMA,
        ],
        compiler_params=pltpu.CompilerParams(use_tc_tiling_on_sc=False),
    )
    return f(table, seq2d, pos2d, neg2d)


def kernel(seq_ids, pos_ids, neg_ids, item_embedding_table):
    seq2d = seq_ids.reshape(N // IDXW, IDXW)
    pos2d = pos_ids.reshape(N // IDXW, IDXW)
    neg2d = neg_ids.reshape(N // IDXW, IDXW)
    seq_emb, pos_emb, neg_emb, istarget = _sc_call(
        item_embedding_table, seq2d, pos2d, neg2d)
    return seq_emb, pos_emb, neg_emb, istarget


# speedup vs baseline: 1.1565x; 1.0431x over previous
import jax
import jax.numpy as jnp
from jax import lax
from jax.experimental import pallas as pl
from jax.experimental.pallas import tpu as pltpu
from jax.experimental.pallas import tpu_sc as plsc

B, L, V, D = 4096, 200, 1000000, 64
N = B * L
NW = 32
PER_W = N // NW
IDXW = 128
CHUNK_IR = 4
CHUNK = CHUNK_IR * IDXW
NCHUNK = PER_W // CHUNK
IR_PER_W = PER_W // IDXW
SCALE = float(D) ** 0.5
LANES = 16
GPR = IDXW // LANES


def _sc_body(table, seq2d, pos2d, neg2d,
             seq_out, pos_out, neg_out, ist_out,
             idx_v0, idx_v1, rows_v0, rows_v1, ist_v0, ist_v1, zacc_v,
             gsem0, gsem1, osem0, osem1):
    wid = lax.axis_index("s") * 2 + lax.axis_index("c")
    idx_vs = (idx_v0, idx_v1)
    rows_vs = (rows_v0, rows_v1)
    ist_vs = (ist_v0, ist_v1)
    gsems = (gsem0, gsem1)
    osems = (osem0, osem1)

    def load_group(b, k):
        j = k // GPR
        off = (k % GPR) * LANES
        return idx_vs[b][j, pl.ds(off, LANES)]

    def run_array(idx2d, out_hbm, scaled, want_ist):
        def fetch(g, b):
            ir = wid * IR_PER_W + g * CHUNK_IR
            pltpu.sync_copy(idx2d.at[pl.ds(ir, CHUNK_IR)], idx_vs[b])
            for j in range(CHUNK_IR):
                pltpu.async_copy(
                    table.at[idx_vs[b].at[j]],
                    rows_vs[b].at[pl.ds(j * IDXW, IDXW)],
                    gsems[b],
                )

        def wait_gather(b):
            pltpu.make_async_copy(
                table.at[pl.ds(0, CHUNK)], rows_vs[b], gsems[b]).wait()

        def fire_out(g, b):
            base = wid * PER_W + g * CHUNK
            pltpu.async_copy(rows_vs[b], out_hbm.at[pl.ds(base, CHUNK)],
                             osems[b])
            if want_ist:
                pltpu.async_copy(ist_vs[b], ist_out.at[pl.ds(base, CHUNK)],
                                 osems[b])

        def wait_out(b):
            pltpu.make_async_copy(
                rows_vs[b], out_hbm.at[pl.ds(0, CHUNK)], osems[b]).wait()
            if want_ist:
                pltpu.make_async_copy(
                    ist_vs[b], ist_out.at[pl.ds(0, CHUNK)], osems[b]).wait()

        def process(b):
            if scaled:
                def scale_group(k, c2):
                    gbase = k * LANES
                    v = load_group(b, k)
                    m = jnp.where(v != 0, jnp.float32(SCALE),
                                  jnp.float32(0.0))
                    for lane in range(LANES):
                        s = m[lane]
                        for cc in range(D // LANES):
                            sl = (gbase + lane, pl.ds(cc * LANES, LANES))
                            rows_vs[b][sl] = rows_vs[b][sl] * s
                    return c2
                lax.fori_loop(0, CHUNK // LANES, scale_group, 0)
            else:
                zacc_v[...] = jnp.zeros((LANES,), jnp.int32)

                def group_body(k, c2):
                    v = load_group(b, k)
                    z = jnp.where(v == 0, jnp.int32(1), jnp.int32(0))
                    zacc_v[...] = zacc_v[...] | z
                    if want_ist:
                        ist_vs[b][pl.ds(k * LANES, LANES)] = jnp.where(
                            v != 0, jnp.float32(1.0), jnp.float32(0.0))
                    return c2
                lax.fori_loop(0, CHUNK // LANES, group_body, 0)

                zacc = zacc_v[...]
                anyz = zacc[0]
                for lane in range(1, LANES):
                    anyz = anyz | zacc[lane]

                @pl.when(anyz != 0)
                def _zero_fix():
                    def fix_group(k, c2):
                        gbase = k * LANES
                        v = load_group(b, k)
                        for lane in range(LANES):
                            @pl.when(v[lane] == 0)
                            def _zero_row():
                                for cc in range(D // LANES):
                                    rows_vs[b][gbase + lane,
                                               pl.ds(cc * LANES, LANES)] = (
                                        jnp.zeros((LANES,), jnp.float32))
                        return c2
                    lax.fori_loop(0, CHUNK // LANES, fix_group, 0)

        fetch(0, 0)

        def pair_body(g2, carry):
            for b in range(2):
                g = g2 * 2 + b

                @pl.when(g + 1 < NCHUNK)
                def _prefetch():
                    @pl.when(g >= 1)
                    def _reuse_wait():
                        wait_out(1 - b)
                    fetch(g + 1, 1 - b)

                wait_gather(b)
                process(b)
                fire_out(g, b)
            return carry

        lax.fori_loop(0, NCHUNK // 2, pair_body, 0)
        wait_out(0)
        wait_out(1)

    run_array(seq2d, seq_out, True, False)
    run_array(pos2d, pos_out, False, True)
    run_array(neg2d, neg_out, False, False)


@jax.jit
def _sc_call(table, seq2d, pos2d, neg2d):
    mesh = plsc.VectorSubcoreMesh(core_axis_name="c", subcore_axis_name="s")
    f = pl.kernel(
        _sc_body,
        out_type=(
            jax.ShapeDtypeStruct((N, D), jnp.float32),
            jax.ShapeDtypeStruct((N, D), jnp.float32),
            jax.ShapeDtypeStruct((N, D), jnp.float32),
            jax.ShapeDtypeStruct((N,), jnp.float32),
        ),
        mesh=mesh,
        scratch_types=[
            pltpu.VMEM((CHUNK_IR, IDXW), jnp.int32),
            pltpu.VMEM((CHUNK_IR, IDXW), jnp.int32),
            pltpu.VMEM((CHUNK, D), jnp.float32),
            pltpu.VMEM((CHUNK, D), jnp.float32),
            pltpu.VMEM((CHUNK,), jnp.float32),
            pltpu.VMEM((CHUNK,), jnp.float32),
            pltpu.VMEM((LANES,), jnp.int32),
            pltpu.SemaphoreType.DMA,
            pltpu.SemaphoreType.DMA,
            pltpu.SemaphoreType.DMA,
            pltpu.SemaphoreType.DMA,
        ],
        compiler_params=pltpu.CompilerParams(use_tc_tiling_on_sc=False),
    )
    return f(table, seq2d, pos2d, neg2d)


def kernel(seq_ids, pos_ids, neg_ids, item_embedding_table):
    seq2d = seq_ids.reshape(N // IDXW, IDXW)
    pos2d = pos_ids.reshape(N // IDXW, IDXW)
    neg2d = neg_ids.reshape(N // IDXW, IDXW)
    seq_emb, pos_emb, neg_emb, istarget = _sc_call(
        item_embedding_table, seq2d, pos2d, neg2d)
    return seq_emb, pos_emb, neg_emb, istarget
